# Initial kernel scaffold; baseline (speedup 1.0000x reference)
import jax
import jax.numpy as jnp
from jax.experimental import pallas as pl


def kernel(node_s, node_v, edge_s, edge_v, edge_index, frames, norm_gamma, norm_beta, g0_wd, g0_ws, g0_wn, g0_bs, g0_wu, g0_wg, g0_bg, g1_wd, g1_ws, g1_wn, g1_bs, g1_wu, g1_wg, g1_bg, attn_w, attn_b, ff_wd, ff_ws, ff_wn, ff_bs, ff_wu, ff_wg, ff_bg, res_w, res_b):
    raise NotImplementedError("write your pallas kernel here")


# speedup vs baseline: 1.4021x; 1.4021x over previous
from functools import partial

import jax
import jax.numpy as jnp
from jax.experimental import pallas as pl
from jax.experimental.pallas import tpu as pltpu

_NODE_TILE = 512
_EDGE_TILE = 1024
_WIN = 512
_SHIFT = 17
_MASK = (1 << _SHIFT) - 1


def _norm_body(s_ref, v_ref, g_ref, b_ref, o_ref, *, nchan):
    s = s_ref[...]
    mu = jnp.mean(s, axis=-1, keepdims=True)
    var = jnp.mean(jnp.square(s - mu), axis=-1, keepdims=True)
    sn = (s - mu) * jax.lax.rsqrt(var + 1e-5) * g_ref[...] + b_ref[...]
    v = v_ref[...]
    msq = jnp.sum(v * v, axis=-1, keepdims=True) / nchan
    vn = v * jax.lax.rsqrt(msq + 1e-8)
    o_ref[...] = jnp.concatenate([sn, vn], axis=-1)


def _gcp(s_parts, vx, vy, vz, wdu, wsn, bs, wg, bg, h, nonlin):
    t = vx.shape[0]
    v_st = jnp.concatenate([vx, vy, vz], axis=0)
    vdu = jnp.dot(v_st, wdu, preferred_element_type=jnp.float32)
    vd, vu = vdu[:, :h], vdu[:, h:]
    vnorm = jnp.sqrt(vd[:t] * vd[:t] + vd[t:2 * t] * vd[t:2 * t]
                     + vd[2 * t:] * vd[2 * t:] + 1e-8)
    s_in = jnp.concatenate(list(s_parts) + [vnorm], axis=-1)
    so = jnp.dot(s_in, wsn, preferred_element_type=jnp.float32) + bs
    gate = jax.nn.sigmoid(jnp.dot(so, wg, preferred_element_type=jnp.float32) + bg)
    vox = vu[:t] * gate
    voy = vu[t:2 * t] * gate
    voz = vu[2 * t:] * gate
    if nonlin:
        so = so * jax.nn.sigmoid(so)
    return so, vox, voy, voz


def _msg_scatter_body(tb_ref, tw_ref, row_ref, rf_ref, cf_ref, es_ref, ev_ref,
                      wdu0_ref, wsn0_ref, bs0_ref, wg0_ref, bg0_ref,
                      wdu1_ref, wsn1_ref, bs1_ref, wg1_ref, bg1_ref,
                      wa_ref, ba_ref, out_ref, *, ns, nv, ev, win):
    c = pl.program_id(0)
    j = pl.program_id(1)
    t = c * pl.num_programs(1) + j

    @pl.when(j == 0)
    def _():
        out_ref[...] = jnp.zeros_like(out_ref)

    rf = rf_ref[...]
    cf = cf_ref[...]
    evs = ev_ref[...]

    def vcomp(k):
        return jnp.concatenate(
            [rf[:, ns + k * nv:ns + (k + 1) * nv],
             evs[:, k * ev:(k + 1) * ev],
             cf[:, ns + k * nv:ns + (k + 1) * nv]], axis=-1)

    s0, v0x, v0y, v0z = _gcp(
        (rf[:, :ns], es_ref[...], cf[:, :ns]), vcomp(0), vcomp(1), vcomp(2),
        wdu0_ref[...], wsn0_ref[...], bs0_ref[...], wg0_ref[...], bg0_ref[...],
        h=2 * nv + ev, nonlin=True)
    s1, v1x, v1y, v1z = _gcp(
        (s0,), v0x, v0y, v0z,
        wdu1_ref[...], wsn1_ref[...], bs1_ref[...], wg1_ref[...], bg1_ref[...],
        h=v0x.shape[1], nonlin=True)

    ms = s0 + s1
    logit = jnp.sum(ms * wa_ref[...], axis=-1, keepdims=True) + ba_ref[...]
    ms = ms * jax.nn.sigmoid(logit)
    msg = jnp.concatenate([ms, v0x + v1x, v0y + v1y, v0z + v1z], axis=-1)

    rows = row_ref[0]
    base0 = tb_ref[t]
    iota = jax.lax.broadcasted_iota(jnp.int32, (win, rows.shape[1]), 0)

    def body(w, _):
        base = pl.multiple_of(base0 + w * win, win)
        onehot = (iota + base == rows).astype(jnp.float32)
        contrib = jnp.dot(onehot, msg, preferred_element_type=jnp.float32)
        out_ref[0, pl.ds(base, win), :] += contrib
        return 0

    jax.lax.fori_loop(0, tw_ref[t], body, 0)


def _node_body(a0_ref, a1_ref, nc_ref,
               wduf_ref, wsnf_ref, bsf_ref, wgf_ref, bgf_ref,
               wr_ref, br_ref, os_ref, ov_ref, *, ns, nv, so_dim, vo_dim):
    agg = a0_ref[0] + a1_ref[0]
    h_s = agg[:, :so_dim]
    h_v = agg[:, so_dim:]
    nc = nc_ref[...]
    sn = nc[:, :ns]
    vn = nc[:, ns:]

    def vcomp(k):
        return jnp.concatenate([h_v[:, k * vo_dim:(k + 1) * vo_dim],
                                vn[:, k * nv:(k + 1) * nv]], axis=-1)

    ff_s, fvx, fvy, fvz = _gcp(
        (h_s, sn), vcomp(0), vcomp(1), vcomp(2),
        wduf_ref[...], wsnf_ref[...], bsf_ref[...], wgf_ref[...], bgf_ref[...],
        h=nv + vo_dim, nonlin=False)

    rs = jnp.concatenate([sn, ff_s], axis=-1)
    os_ref[...] = jnp.dot(rs, wr_ref[...],
                          preferred_element_type=jnp.float32) + br_ref[...]
    ov_ref[...] = jnp.concatenate(
        [vn[:, :nv] + fvx, vn[:, nv:2 * nv] + fvy, vn[:, 2 * nv:] + fvz],
        axis=-1)


def _rep2(shape):
    return pl.BlockSpec(shape, lambda *a: (0, 0))


def kernel(node_s, node_v, edge_s, edge_v, edge_index, frames, norm_gamma,
           norm_beta, g0_wd, g0_ws, g0_wn, g0_bs, g0_wu, g0_wg, g0_bg,
           g1_wd, g1_ws, g1_wn, g1_bs, g1_wu, g1_wg, g1_bg, attn_w, attn_b,
           ff_wd, ff_ws, ff_wn, ff_bs, ff_wu, ff_wg, ff_bg, res_w, res_b):
    N, nv = node_v.shape[0], node_v.shape[1]
    E, ev = edge_v.shape[0], edge_v.shape[1]
    ns = node_s.shape[1]
    es = edge_s.shape[1]
    so_dim = g0_ws.shape[1]
    vo_dim = g0_wu.shape[1]
    d_msg = so_dim + 3 * vo_dim

    tn = min(_NODE_TILE, N)
    te = min(_EDGE_TILE, E)
    n_tiles = E // te
    nj = n_tiles // 2
    win = min(_WIN, N)

    node_v2 = jnp.transpose(node_v, (0, 2, 1)).reshape(N, 3 * nv)
    edge_v2 = jnp.transpose(edge_v, (0, 2, 1)).reshape(E, 3 * ev)

    ncat = pl.pallas_call(
        partial(_norm_body, nchan=nv),
        grid=(N // tn,),
        in_specs=[pl.BlockSpec((tn, ns), lambda i: (i, 0)),
                  pl.BlockSpec((tn, 3 * nv), lambda i: (i, 0)),
                  _rep2((1, ns)), _rep2((1, ns))],
        out_specs=pl.BlockSpec((tn, ns + 3 * nv), lambda i: (i, 0)),
        out_shape=jax.ShapeDtypeStruct((N, ns + 3 * nv), jnp.float32),
        compiler_params=pltpu.CompilerParams(dimension_semantics=("parallel",)),
    )(node_s, node_v2, norm_gamma, norm_beta)

    row = edge_index[0].astype(jnp.int32)
    col = edge_index[1].astype(jnp.int32)
    key = (row << _SHIFT) | jax.lax.iota(jnp.int32, E)
    skey = jax.lax.sort(key)
    srow = skey >> _SHIFT
    perm = skey & _MASK
    scol = jnp.take(col, perm, axis=0)

    w_start = srow[0::te] // win
    w_end = srow[te - 1::te] // win
    tile_base = (w_start * win).astype(jnp.int32)
    tile_nw = (w_end - w_start + 1).astype(jnp.int32)

    row_feat = jnp.take(ncat, srow, axis=0)
    col_feat = jnp.take(ncat, scol, axis=0)
    edge_s_s = jnp.take(edge_s, perm, axis=0)
    edge_v_s = jnp.take(edge_v2, perm, axis=0)
    srow3 = srow.reshape(n_tiles, 1, te)

    g0_wdu = jnp.concatenate([g0_wd, g0_wd @ g0_wu], axis=1)
    g0_wsn = jnp.concatenate([g0_ws, g0_wn], axis=0)
    g1_wdu = jnp.concatenate([g1_wd, g1_wd @ g1_wu], axis=1)
    g1_wsn = jnp.concatenate([g1_ws, g1_wn], axis=0)
    ff_wdu = jnp.concatenate([ff_wd, ff_wd @ ff_wu], axis=1)
    ff_wsn = jnp.concatenate([ff_ws, ff_wn], axis=0)

    h0 = 2 * nv + ev
    tile_map = lambda c, j, *_: (c * nj + j, 0)

    agg2 = pl.pallas_call(
        partial(_msg_scatter_body, ns=ns, nv=nv, ev=ev, win=win),
        grid_spec=pltpu.PrefetchScalarGridSpec(
            num_scalar_prefetch=2,
            grid=(2, nj),
            in_specs=[
                pl.BlockSpec((1, 1, te), lambda c, j, *_: (c * nj + j, 0, 0)),
                pl.BlockSpec((te, ns + 3 * nv), tile_map),
                pl.BlockSpec((te, ns + 3 * nv), tile_map),
                pl.BlockSpec((te, es), tile_map),
                pl.BlockSpec((te, 3 * ev), tile_map),
                _rep2((h0, h0 + vo_dim)),
                _rep2((2 * ns + es + h0, so_dim)),
                _rep2((1, so_dim)),
                _rep2((so_dim, vo_dim)),
                _rep2((1, vo_dim)),
                _rep2((vo_dim, 2 * vo_dim)),
                _rep2((so_dim + vo_dim, so_dim)),
                _rep2((1, so_dim)),
                _rep2((so_dim, vo_dim)),
                _rep2((1, vo_dim)),
                _rep2((1, so_dim)),
                _rep2((1, 1)),
            ],
            out_specs=pl.BlockSpec((1, N, d_msg), lambda c, j, *_: (c, 0, 0)),
        ),
        out_shape=jax.ShapeDtypeStruct((2, N, d_msg), jnp.float32),
        compiler_params=pltpu.CompilerParams(
            dimension_semantics=("parallel", "arbitrary")),
    )(tile_base, tile_nw, srow3, row_feat, col_feat, edge_s_s, edge_v_s,
      g0_wdu, g0_wsn, g0_bs, g0_wg, g0_bg,
      g1_wdu, g1_wsn, g1_bs, g1_wg, g1_bg, attn_w, attn_b)

    out_s, out_vf = pl.pallas_call(
        partial(_node_body, ns=ns, nv=nv, so_dim=so_dim, vo_dim=vo_dim),
        grid=(N // tn,),
        in_specs=[
            pl.BlockSpec((1, tn, d_msg), lambda i: (0, i, 0)),
            pl.BlockSpec((1, tn, d_msg), lambda i: (1, i, 0)),
            pl.BlockSpec((tn, ns + 3 * nv), lambda i: (i, 0)),
            _rep2((nv + vo_dim, nv + 2 * vo_dim)),
            _rep2((ns + so_dim + nv + vo_dim, so_dim)),
            _rep2((1, so_dim)),
            _rep2((so_dim, vo_dim)),
            _rep2((1, vo_dim)),
            _rep2((ns + so_dim, so_dim)),
            _rep2((1, so_dim)),
        ],
        out_specs=(pl.BlockSpec((tn, so_dim), lambda i: (i, 0)),
                   pl.BlockSpec((tn, 3 * vo_dim), lambda i: (i, 0))),
        out_shape=(jax.ShapeDtypeStruct((N, so_dim), jnp.float32),
                   jax.ShapeDtypeStruct((N, 3 * vo_dim), jnp.float32)),
        compiler_params=pltpu.CompilerParams(dimension_semantics=("parallel",)),
    )(agg2, agg2, ncat, ff_wdu, ff_wsn, ff_bs, ff_wg, ff_bg, res_w, res_b)

    out_v = jnp.transpose(out_vf.reshape(N, 3, vo_dim), (0, 2, 1))
    return out_s, out_v


# speedup vs baseline: 2.0220x; 1.4421x over previous
from functools import partial

import jax
import jax.numpy as jnp
from jax.experimental import pallas as pl
from jax.experimental.pallas import tpu as pltpu

_NODE_TILE = 512
_EDGE_TILE = 1024
_WIN = 512
_SHIFT = 17
_MASK = (1 << _SHIFT) - 1


def _norm_body(s_ref, v_ref, g_ref, b_ref, o_ref, *, nchan):
    s = s_ref[...]
    mu = jnp.mean(s, axis=-1, keepdims=True)
    var = jnp.mean(jnp.square(s - mu), axis=-1, keepdims=True)
    sn = (s - mu) * jax.lax.rsqrt(var + 1e-5) * g_ref[...] + b_ref[...]
    v = v_ref[...]
    msq = jnp.sum(v * v, axis=-1, keepdims=True) / nchan
    vn = v * jax.lax.rsqrt(msq + 1e-8)
    o_ref[...] = jnp.concatenate([sn, vn], axis=-1)


def _gcp(s_parts, vx, vy, vz, wdu, wsn, bs, wg, bg, h, nonlin):
    t = vx.shape[0]
    v_st = jnp.concatenate([vx, vy, vz], axis=0)
    vdu = jnp.dot(v_st, wdu, preferred_element_type=jnp.float32)
    vd, vu = vdu[:, :h], vdu[:, h:]
    vnorm = jnp.sqrt(vd[:t] * vd[:t] + vd[t:2 * t] * vd[t:2 * t]
                     + vd[2 * t:] * vd[2 * t:] + 1e-8)
    s_in = jnp.concatenate(list(s_parts) + [vnorm], axis=-1)
    so = jnp.dot(s_in, wsn, preferred_element_type=jnp.float32) + bs
    gate = jax.nn.sigmoid(jnp.dot(so, wg, preferred_element_type=jnp.float32) + bg)
    vox = vu[:t] * gate
    voy = vu[t:2 * t] * gate
    voz = vu[2 * t:] * gate
    if nonlin:
        so = so * jax.nn.sigmoid(so)
    return so, vox, voy, voz


def _msg_scatter_body(tb_ref, tw_ref, row_ref, nc_ref, cf_ref, es_ref, ev_ref,
                      wdu0_ref, wsn0_ref, bs0_ref, wg0_ref, bg0_ref,
                      wdu1_ref, wsn1_ref, bs1_ref, wg1_ref, bg1_ref,
                      wa_ref, ba_ref, out_ref, *, ns, nv, ev, win):
    c = pl.program_id(0)
    j = pl.program_id(1)
    t = c * pl.num_programs(1) + j

    @pl.when(j == 0)
    def _():
        out_ref[...] = jnp.zeros_like(out_ref)

    rows = row_ref[0]
    te = rows.shape[1]
    base0 = tb_ref[t]
    nw = tw_ref[t]
    iota = jax.lax.broadcasted_iota(jnp.int32, (win, te), 0)
    d_node = nc_ref.shape[1]

    def gbody(w, acc):
        base = pl.multiple_of(base0 + w * win, win)
        onehot = (iota + base == rows).astype(jnp.float32)
        band = nc_ref[pl.ds(base, win), :]
        return acc + jax.lax.dot_general(
            onehot, band, (((0,), (0,)), ((), ())),
            preferred_element_type=jnp.float32)

    rf = jax.lax.fori_loop(0, nw, gbody,
                           jnp.zeros((te, d_node), jnp.float32))
    cf = cf_ref[...]
    evs = ev_ref[...]

    def vcomp(k):
        return jnp.concatenate(
            [rf[:, ns + k * nv:ns + (k + 1) * nv],
             evs[:, k * ev:(k + 1) * ev],
             cf[:, ns + k * nv:ns + (k + 1) * nv]], axis=-1)

    s0, v0x, v0y, v0z = _gcp(
        (rf[:, :ns], es_ref[...], cf[:, :ns]), vcomp(0), vcomp(1), vcomp(2),
        wdu0_ref[...], wsn0_ref[...], bs0_ref[...], wg0_ref[...], bg0_ref[...],
        h=2 * nv + ev, nonlin=True)
    s1, v1x, v1y, v1z = _gcp(
        (s0,), v0x, v0y, v0z,
        wdu1_ref[...], wsn1_ref[...], bs1_ref[...], wg1_ref[...], bg1_ref[...],
        h=v0x.shape[1], nonlin=True)

    ms = s0 + s1
    logit = jnp.sum(ms * wa_ref[...], axis=-1, keepdims=True) + ba_ref[...]
    ms = ms * jax.nn.sigmoid(logit)
    msg = jnp.concatenate([ms, v0x + v1x, v0y + v1y, v0z + v1z], axis=-1)

    def body(w, _):
        base = pl.multiple_of(base0 + w * win, win)
        onehot = (iota + base == rows).astype(jnp.float32)
        contrib = jnp.dot(onehot, msg, preferred_element_type=jnp.float32)
        out_ref[0, pl.ds(base, win), :] += contrib
        return 0

    jax.lax.fori_loop(0, nw, body, 0)


def _node_body(a0_ref, a1_ref, nc_ref,
               wduf_ref, wsnf_ref, bsf_ref, wgf_ref, bgf_ref,
               wr_ref, br_ref, os_ref, ov_ref, *, ns, nv, so_dim, vo_dim):
    agg = a0_ref[0] + a1_ref[0]
    h_s = agg[:, :so_dim]
    h_v = agg[:, so_dim:]
    nc = nc_ref[...]
    sn = nc[:, :ns]
    vn = nc[:, ns:]

    def vcomp(k):
        return jnp.concatenate([h_v[:, k * vo_dim:(k + 1) * vo_dim],
                                vn[:, k * nv:(k + 1) * nv]], axis=-1)

    ff_s, fvx, fvy, fvz = _gcp(
        (h_s, sn), vcomp(0), vcomp(1), vcomp(2),
        wduf_ref[...], wsnf_ref[...], bsf_ref[...], wgf_ref[...], bgf_ref[...],
        h=nv + vo_dim, nonlin=False)

    rs = jnp.concatenate([sn, ff_s], axis=-1)
    os_ref[...] = jnp.dot(rs, wr_ref[...],
                          preferred_element_type=jnp.float32) + br_ref[...]
    ov_ref[...] = jnp.concatenate(
        [vn[:, :nv] + fvx, vn[:, nv:2 * nv] + fvy, vn[:, 2 * nv:] + fvz],
        axis=-1)


def _rep2(shape):
    return pl.BlockSpec(shape, lambda *a: (0, 0))


def kernel(node_s, node_v, edge_s, edge_v, edge_index, frames, norm_gamma,
           norm_beta, g0_wd, g0_ws, g0_wn, g0_bs, g0_wu, g0_wg, g0_bg,
           g1_wd, g1_ws, g1_wn, g1_bs, g1_wu, g1_wg, g1_bg, attn_w, attn_b,
           ff_wd, ff_ws, ff_wn, ff_bs, ff_wu, ff_wg, ff_bg, res_w, res_b):
    N, nv = node_v.shape[0], node_v.shape[1]
    E, ev = edge_v.shape[0], edge_v.shape[1]
    ns = node_s.shape[1]
    es = edge_s.shape[1]
    so_dim = g0_ws.shape[1]
    vo_dim = g0_wu.shape[1]
    d_msg = so_dim + 3 * vo_dim

    tn = min(_NODE_TILE, N)
    te = min(_EDGE_TILE, E)
    n_tiles = E // te
    nj = n_tiles // 2
    win = min(_WIN, N)

    node_v2 = jnp.transpose(node_v, (0, 2, 1)).reshape(N, 3 * nv)
    edge_v2 = jnp.transpose(edge_v, (0, 2, 1)).reshape(E, 3 * ev)

    ncat = pl.pallas_call(
        partial(_norm_body, nchan=nv),
        grid=(N // tn,),
        in_specs=[pl.BlockSpec((tn, ns), lambda i: (i, 0)),
                  pl.BlockSpec((tn, 3 * nv), lambda i: (i, 0)),
                  _rep2((1, ns)), _rep2((1, ns))],
        out_specs=pl.BlockSpec((tn, ns + 3 * nv), lambda i: (i, 0)),
        out_shape=jax.ShapeDtypeStruct((N, ns + 3 * nv), jnp.float32),
        compiler_params=pltpu.CompilerParams(dimension_semantics=("parallel",)),
    )(node_s, node_v2, norm_gamma, norm_beta)

    row = edge_index[0].astype(jnp.int32)
    col = edge_index[1].astype(jnp.int32)
    key = (row << _SHIFT) | jax.lax.iota(jnp.int32, E)
    skey = jax.lax.sort(key)
    srow = skey >> _SHIFT
    perm = skey & _MASK
    scol = col.at[perm].get(mode="promise_in_bounds")

    w_start = srow[0::te] // win
    w_end = srow[te - 1::te] // win
    tile_base = (w_start * win).astype(jnp.int32)
    tile_nw = (w_end - w_start + 1).astype(jnp.int32)

    col_feat = ncat.at[scol].get(mode="promise_in_bounds")
    edge_s_s = edge_s.at[perm].get(mode="promise_in_bounds")
    edge_v_s = edge_v2.at[perm].get(mode="promise_in_bounds")
    srow3 = srow.reshape(n_tiles, 1, te)

    g0_wdu = jnp.concatenate([g0_wd, g0_wd @ g0_wu], axis=1)
    g0_wsn = jnp.concatenate([g0_ws, g0_wn], axis=0)
    g1_wdu = jnp.concatenate([g1_wd, g1_wd @ g1_wu], axis=1)
    g1_wsn = jnp.concatenate([g1_ws, g1_wn], axis=0)
    ff_wdu = jnp.concatenate([ff_wd, ff_wd @ ff_wu], axis=1)
    ff_wsn = jnp.concatenate([ff_ws, ff_wn], axis=0)

    h0 = 2 * nv + ev
    tile_map = lambda c, j, *_: (c * nj + j, 0)

    agg2 = pl.pallas_call(
        partial(_msg_scatter_body, ns=ns, nv=nv, ev=ev, win=win),
        grid_spec=pltpu.PrefetchScalarGridSpec(
            num_scalar_prefetch=2,
            grid=(2, nj),
            in_specs=[
                pl.BlockSpec((1, 1, te), lambda c, j, *_: (c * nj + j, 0, 0)),
                pl.BlockSpec((N, ns + 3 * nv), lambda c, j, *_: (0, 0)),
                pl.BlockSpec((te, ns + 3 * nv), tile_map),
                pl.BlockSpec((te, es), tile_map),
                pl.BlockSpec((te, 3 * ev), tile_map),
                _rep2((h0, h0 + vo_dim)),
                _rep2((2 * ns + es + h0, so_dim)),
                _rep2((1, so_dim)),
                _rep2((so_dim, vo_dim)),
                _rep2((1, vo_dim)),
                _rep2((vo_dim, 2 * vo_dim)),
                _rep2((so_dim + vo_dim, so_dim)),
                _rep2((1, so_dim)),
                _rep2((so_dim, vo_dim)),
                _rep2((1, vo_dim)),
                _rep2((1, so_dim)),
                _rep2((1, 1)),
            ],
            out_specs=pl.BlockSpec((1, N, d_msg), lambda c, j, *_: (c, 0, 0)),
        ),
        out_shape=jax.ShapeDtypeStruct((2, N, d_msg), jnp.float32),
        compiler_params=pltpu.CompilerParams(
            dimension_semantics=("parallel", "arbitrary")),
    )(tile_base, tile_nw, srow3, ncat, col_feat, edge_s_s, edge_v_s,
      g0_wdu, g0_wsn, g0_bs, g0_wg, g0_bg,
      g1_wdu, g1_wsn, g1_bs, g1_wg, g1_bg, attn_w, attn_b)

    out_s, out_vf = pl.pallas_call(
        partial(_node_body, ns=ns, nv=nv, so_dim=so_dim, vo_dim=vo_dim),
        grid=(N // tn,),
        in_specs=[
            pl.BlockSpec((1, tn, d_msg), lambda i: (0, i, 0)),
            pl.BlockSpec((1, tn, d_msg), lambda i: (1, i, 0)),
            pl.BlockSpec((tn, ns + 3 * nv), lambda i: (i, 0)),
            _rep2((nv + vo_dim, nv + 2 * vo_dim)),
            _rep2((ns + so_dim + nv + vo_dim, so_dim)),
            _rep2((1, so_dim)),
            _rep2((so_dim, vo_dim)),
            _rep2((1, vo_dim)),
            _rep2((ns + so_dim, so_dim)),
            _rep2((1, so_dim)),
        ],
        out_specs=(pl.BlockSpec((tn, so_dim), lambda i: (i, 0)),
                   pl.BlockSpec((tn, 3 * vo_dim), lambda i: (i, 0))),
        out_shape=(jax.ShapeDtypeStruct((N, so_dim), jnp.float32),
                   jax.ShapeDtypeStruct((N, 3 * vo_dim), jnp.float32)),
        compiler_params=pltpu.CompilerParams(dimension_semantics=("parallel",)),
    )(agg2, agg2, ncat, ff_wdu, ff_wsn, ff_bs, ff_wg, ff_bg, res_w, res_b)

    out_v = jnp.transpose(out_vf.reshape(N, 3, vo_dim), (0, 2, 1))
    return out_s, out_v


# speedup vs baseline: 2.4222x; 1.1979x over previous
from functools import partial

import jax
import jax.numpy as jnp
from jax.experimental import pallas as pl
from jax.experimental.pallas import tpu as pltpu

_NODE_TILE = 512
_EDGE_TILE = 1024
_WIN = 512
_SHIFT = 17
_MASK = (1 << _SHIFT) - 1


def _norm_body(s_ref, v_ref, g_ref, b_ref, o_ref, *, nchan):
    s = s_ref[...]
    mu = jnp.mean(s, axis=-1, keepdims=True)
    var = jnp.mean(jnp.square(s - mu), axis=-1, keepdims=True)
    sn = (s - mu) * jax.lax.rsqrt(var + 1e-5) * g_ref[...] + b_ref[...]
    v = v_ref[...]
    msq = jnp.sum(v * v, axis=-1, keepdims=True) / nchan
    vn = v * jax.lax.rsqrt(msq + 1e-8)
    o_ref[...] = jnp.concatenate([sn, vn], axis=-1)


def _gcp(s_parts, vx, vy, vz, wdu, wsn, bs, wg, bg, h, nonlin):
    t = vx.shape[0]
    v_st = jnp.concatenate([vx, vy, vz], axis=0)
    vdu = jnp.dot(v_st, wdu, preferred_element_type=jnp.float32)
    vd, vu = vdu[:, :h], vdu[:, h:]
    vnorm = jnp.sqrt(vd[:t] * vd[:t] + vd[t:2 * t] * vd[t:2 * t]
                     + vd[2 * t:] * vd[2 * t:] + 1e-8)
    s_in = jnp.concatenate(list(s_parts) + [vnorm], axis=-1)
    so = jnp.dot(s_in, wsn, preferred_element_type=jnp.float32) + bs
    gate = jax.nn.sigmoid(jnp.dot(so, wg, preferred_element_type=jnp.float32) + bg)
    vox = vu[:t] * gate
    voy = vu[t:2 * t] * gate
    voz = vu[2 * t:] * gate
    if nonlin:
        so = so * jax.nn.sigmoid(so)
    return so, vox, voy, voz


def _msg_scatter_body(tb_ref, tw_ref, row_ref, col_ref, nc_ref, nc2_ref,
                      es_ref, ev_ref,
                      wdu0_ref, wsn0_ref, bs0_ref, wg0_ref, bg0_ref,
                      wdu1_ref, wsn1_ref, bs1_ref, wg1_ref, bg1_ref,
                      wa_ref, ba_ref, out_ref, cs_ref, cv_ref,
                      *, ns, nv, ev, win):
    c = pl.program_id(0)
    j = pl.program_id(1)
    t = c * pl.num_programs(1) + j

    @pl.when(j == 0)
    def _():
        out_ref[...] = jnp.zeros_like(out_ref)

    rows = row_ref[0]
    te = rows.shape[1]
    base0 = tb_ref[t]
    nw = tw_ref[t]
    iota = jax.lax.broadcasted_iota(jnp.int32, (win, te), 0)
    d_node = nc_ref.shape[1]

    for mi in range(te):
        i2 = pl.multiple_of(col_ref[0, 0, mi] * 2, 2)
        slab = nc2_ref[pl.ds(i2, 2), :]
        cs_ref[mi:mi + 1, :] = slab[0:1, :]
        cv_ref[mi:mi + 1, :] = slab[1:2, :]

    def gbody(w, acc):
        base = pl.multiple_of(base0 + w * win, win)
        onehot = (iota + base == rows).astype(jnp.float32)
        band = nc_ref[pl.ds(base, win), :]
        return acc + jax.lax.dot_general(
            onehot, band, (((0,), (0,)), ((), ())),
            preferred_element_type=jnp.float32)

    rf = jax.lax.fori_loop(0, nw, gbody,
                           jnp.zeros((te, d_node), jnp.float32))
    cs = cs_ref[...]
    cv = cv_ref[...]
    evs = ev_ref[...]

    def vcomp(k):
        return jnp.concatenate(
            [rf[:, ns + k * nv:ns + (k + 1) * nv],
             evs[:, k * ev:(k + 1) * ev],
             cv[:, k * nv:(k + 1) * nv]], axis=-1)

    s0, v0x, v0y, v0z = _gcp(
        (rf[:, :ns], es_ref[...], cs), vcomp(0), vcomp(1), vcomp(2),
        wdu0_ref[...], wsn0_ref[...], bs0_ref[...], wg0_ref[...], bg0_ref[...],
        h=2 * nv + ev, nonlin=True)
    s1, v1x, v1y, v1z = _gcp(
        (s0,), v0x, v0y, v0z,
        wdu1_ref[...], wsn1_ref[...], bs1_ref[...], wg1_ref[...], bg1_ref[...],
        h=v0x.shape[1], nonlin=True)

    ms = s0 + s1
    logit = jnp.sum(ms * wa_ref[...], axis=-1, keepdims=True) + ba_ref[...]
    ms = ms * jax.nn.sigmoid(logit)
    msg = jnp.concatenate([ms, v0x + v1x, v0y + v1y, v0z + v1z], axis=-1)

    def body(w, _):
        base = pl.multiple_of(base0 + w * win, win)
        onehot = (iota + base == rows).astype(jnp.float32)
        contrib = jnp.dot(onehot, msg, preferred_element_type=jnp.float32)
        out_ref[0, pl.ds(base, win), :] += contrib
        return 0

    jax.lax.fori_loop(0, nw, body, 0)


def _node_body(a0_ref, a1_ref, nc_ref,
               wduf_ref, wsnf_ref, bsf_ref, wgf_ref, bgf_ref,
               wr_ref, br_ref, os_ref, ov_ref, *, ns, nv, so_dim, vo_dim):
    agg = a0_ref[0] + a1_ref[0]
    h_s = agg[:, :so_dim]
    h_v = agg[:, so_dim:]
    nc = nc_ref[...]
    sn = nc[:, :ns]
    vn = nc[:, ns:]

    def vcomp(k):
        return jnp.concatenate([h_v[:, k * vo_dim:(k + 1) * vo_dim],
                                vn[:, k * nv:(k + 1) * nv]], axis=-1)

    ff_s, fvx, fvy, fvz = _gcp(
        (h_s, sn), vcomp(0), vcomp(1), vcomp(2),
        wduf_ref[...], wsnf_ref[...], bsf_ref[...], wgf_ref[...], bgf_ref[...],
        h=nv + vo_dim, nonlin=False)

    rs = jnp.concatenate([sn, ff_s], axis=-1)
    os_ref[...] = jnp.dot(rs, wr_ref[...],
                          preferred_element_type=jnp.float32) + br_ref[...]
    ov_ref[...] = jnp.concatenate(
        [vn[:, :nv] + fvx, vn[:, nv:2 * nv] + fvy, vn[:, 2 * nv:] + fvz],
        axis=-1)


def _rep2(shape):
    return pl.BlockSpec(shape, lambda *a: (0, 0))


def kernel(node_s, node_v, edge_s, edge_v, edge_index, frames, norm_gamma,
           norm_beta, g0_wd, g0_ws, g0_wn, g0_bs, g0_wu, g0_wg, g0_bg,
           g1_wd, g1_ws, g1_wn, g1_bs, g1_wu, g1_wg, g1_bg, attn_w, attn_b,
           ff_wd, ff_ws, ff_wn, ff_bs, ff_wu, ff_wg, ff_bg, res_w, res_b):
    N, nv = node_v.shape[0], node_v.shape[1]
    E, ev = edge_v.shape[0], edge_v.shape[1]
    ns = node_s.shape[1]
    es = edge_s.shape[1]
    so_dim = g0_ws.shape[1]
    vo_dim = g0_wu.shape[1]
    d_msg = so_dim + 3 * vo_dim

    tn = min(_NODE_TILE, N)
    te = min(_EDGE_TILE, E)
    n_tiles = E // te
    nj = n_tiles // 2
    win = min(_WIN, N)

    node_v2 = jnp.transpose(node_v, (0, 2, 1)).reshape(N, 3 * nv)
    edge_v2 = jnp.transpose(edge_v, (0, 2, 1)).reshape(E, 3 * ev)

    ncat = pl.pallas_call(
        partial(_norm_body, nchan=nv),
        grid=(N // tn,),
        in_specs=[pl.BlockSpec((tn, ns), lambda i: (i, 0)),
                  pl.BlockSpec((tn, 3 * nv), lambda i: (i, 0)),
                  _rep2((1, ns)), _rep2((1, ns))],
        out_specs=pl.BlockSpec((tn, ns + 3 * nv), lambda i: (i, 0)),
        out_shape=jax.ShapeDtypeStruct((N, ns + 3 * nv), jnp.float32),
        compiler_params=pltpu.CompilerParams(dimension_semantics=("parallel",)),
    )(node_s, node_v2, norm_gamma, norm_beta)

    row = edge_index[0].astype(jnp.int32)
    col = edge_index[1].astype(jnp.int32)
    key = (row << _SHIFT) | jax.lax.iota(jnp.int32, E)
    skey = jax.lax.sort(key)
    srow = skey >> _SHIFT
    perm = skey & _MASK
    scol = col.at[perm].get(mode="promise_in_bounds")

    w_start = srow[0::te] // win
    w_end = srow[te - 1::te] // win
    tile_base = (w_start * win).astype(jnp.int32)
    tile_nw = (w_end - w_start + 1).astype(jnp.int32)

    edge_s_s = edge_s.at[perm].get(mode="promise_in_bounds")
    edge_v_s = edge_v2.at[perm].get(mode="promise_in_bounds")
    srow3 = srow.reshape(n_tiles, 1, te)
    scol3 = scol.reshape(n_tiles, 1, te)
    d_node = ns + 3 * nv
    ncat2 = jnp.pad(ncat, ((0, 0), (0, 256 - d_node))).reshape(2 * N, 128)

    g0_wdu = jnp.concatenate([g0_wd, g0_wd @ g0_wu], axis=1)
    g0_wsn = jnp.concatenate([g0_ws, g0_wn], axis=0)
    g1_wdu = jnp.concatenate([g1_wd, g1_wd @ g1_wu], axis=1)
    g1_wsn = jnp.concatenate([g1_ws, g1_wn], axis=0)
    ff_wdu = jnp.concatenate([ff_wd, ff_wd @ ff_wu], axis=1)
    ff_wsn = jnp.concatenate([ff_ws, ff_wn], axis=0)

    h0 = 2 * nv + ev
    tile_map = lambda c, j, *_: (c * nj + j, 0)

    agg2 = pl.pallas_call(
        partial(_msg_scatter_body, ns=ns, nv=nv, ev=ev, win=win),
        grid_spec=pltpu.PrefetchScalarGridSpec(
            num_scalar_prefetch=2,
            grid=(2, nj),
            in_specs=[
                pl.BlockSpec((1, 1, te), lambda c, j, *_: (c * nj + j, 0, 0)),
                pl.BlockSpec((1, 1, te), lambda c, j, *_: (c * nj + j, 0, 0),
                             memory_space=pltpu.SMEM),
                pl.BlockSpec((N, ns + 3 * nv), lambda c, j, *_: (0, 0)),
                pl.BlockSpec((2 * N, 128), lambda c, j, *_: (0, 0)),
                pl.BlockSpec((te, es), tile_map),
                pl.BlockSpec((te, 3 * ev), tile_map),
                _rep2((h0, h0 + vo_dim)),
                _rep2((2 * ns + es + h0, so_dim)),
                _rep2((1, so_dim)),
                _rep2((so_dim, vo_dim)),
                _rep2((1, vo_dim)),
                _rep2((vo_dim, 2 * vo_dim)),
                _rep2((so_dim + vo_dim, so_dim)),
                _rep2((1, so_dim)),
                _rep2((so_dim, vo_dim)),
                _rep2((1, vo_dim)),
                _rep2((1, so_dim)),
                _rep2((1, 1)),
            ],
            out_specs=pl.BlockSpec((1, N, d_msg), lambda c, j, *_: (c, 0, 0)),
            scratch_shapes=[pltpu.VMEM((te, 128), jnp.float32),
                            pltpu.VMEM((te, 128), jnp.float32)],
        ),
        out_shape=jax.ShapeDtypeStruct((2, N, d_msg), jnp.float32),
        compiler_params=pltpu.CompilerParams(
            dimension_semantics=("parallel", "arbitrary")),
    )(tile_base, tile_nw, srow3, scol3, ncat, ncat2, edge_s_s, edge_v_s,
      g0_wdu, g0_wsn, g0_bs, g0_wg, g0_bg,
      g1_wdu, g1_wsn, g1_bs, g1_wg, g1_bg, attn_w, attn_b)

    out_s, out_vf = pl.pallas_call(
        partial(_node_body, ns=ns, nv=nv, so_dim=so_dim, vo_dim=vo_dim),
        grid=(N // tn,),
        in_specs=[
            pl.BlockSpec((1, tn, d_msg), lambda i: (0, i, 0)),
            pl.BlockSpec((1, tn, d_msg), lambda i: (1, i, 0)),
            pl.BlockSpec((tn, ns + 3 * nv), lambda i: (i, 0)),
            _rep2((nv + vo_dim, nv + 2 * vo_dim)),
            _rep2((ns + so_dim + nv + vo_dim, so_dim)),
            _rep2((1, so_dim)),
            _rep2((so_dim, vo_dim)),
            _rep2((1, vo_dim)),
            _rep2((ns + so_dim, so_dim)),
            _rep2((1, so_dim)),
        ],
        out_specs=(pl.BlockSpec((tn, so_dim), lambda i: (i, 0)),
                   pl.BlockSpec((tn, 3 * vo_dim), lambda i: (i, 0))),
        out_shape=(jax.ShapeDtypeStruct((N, so_dim), jnp.float32),
                   jax.ShapeDtypeStruct((N, 3 * vo_dim), jnp.float32)),
        compiler_params=pltpu.CompilerParams(dimension_semantics=("parallel",)),
    )(agg2, agg2, ncat, ff_wdu, ff_wsn, ff_bs, ff_wg, ff_bg, res_w, res_b)

    out_v = jnp.transpose(out_vf.reshape(N, 3, vo_dim), (0, 2, 1))
    return out_s, out_v
